# baseline (speedup 1.0000x reference)
import jax
import jax.numpy as jnp
from jax.experimental import pallas as pl

_B_BLK = 16


def _body(item_ref, hist_ref, ctx_ref, out_ref):
    item = item_ref[...]
    ctx = ctx_ref[...]
    h = hist_ref[...]
    B, L = h.shape[0], h.shape[1]
    item_p = jnp.broadcast_to(item[:, None, :, None, :], (B, L, 3, 9, 16)).reshape(B, L, 27, 16)
    ctx_p = jnp.broadcast_to(ctx[:, None, None, :, :], (B, L, 9, 3, 16)).reshape(B, L, 27, 16)
    hist_p = jnp.broadcast_to(h[:, :, None, :, None, :], (B, L, 3, 3, 3, 16)).reshape(B, L, 27, 16)
    out_ref[...] = jnp.concatenate([item_p, ctx_p, hist_p], axis=-1)


def kernel(x_item, hist, x_context):
    N, L = hist.shape[0], hist.shape[1]
    return pl.pallas_call(
        _body,
        grid=(N // _B_BLK,),
        in_specs=[
            pl.BlockSpec((_B_BLK, 3, 16), lambda b: (b, 0, 0)),
            pl.BlockSpec((_B_BLK, L, 3, 16), lambda b: (b, 0, 0, 0)),
            pl.BlockSpec((_B_BLK, 3, 16), lambda b: (b, 0, 0)),
        ],
        out_specs=pl.BlockSpec((_B_BLK, L, 27, 48), lambda b: (b, 0, 0, 0)),
        out_shape=jax.ShapeDtypeStruct((N, L, 27, 48), x_item.dtype),
    )(x_item, hist, x_context)
